# baseline (speedup 1.0000x reference)
import jax
import jax.numpy as jnp
from jax import lax
from jax.experimental import pallas as pl
from jax.experimental.pallas import tpu as pltpu
from jax.experimental.pallas import tpu_sc as plsc

_N_FIELDS = 26
_VOCAB = 1000
_EMB = 128
_BATCH = 16384

_NC = 2
_NS = 16
_NW = _NC * _NS
_BPW = _BATCH // _NW
_CHUNK = 64
_NBUF = 8
_NQ = _BPW // _CHUNK


def _sc_body(xt_hbm, tab_hbm, out_hbm, idxv, *rest):
    bufs = rest[:_NBUF]
    ssem = rest[_NBUF]
    gsem = rest[_NBUF + 1:2 * _NBUF + 1]
    wsem = rest[2 * _NBUF + 1:]

    wid = lax.axis_index("s") * _NC + lax.axis_index("c")
    row0 = pl.multiple_of(wid * _BPW, _BPW)

    for i in range(_N_FIELDS):
        pltpu.async_copy(xt_hbm.at[pl.ds(i * _BATCH + row0, _BPW)],
                         idxv.at[i], ssem)
    for i in range(_N_FIELDS):
        pltpu.make_async_copy(xt_hbm.at[pl.ds(0, _BPW)], idxv.at[0],
                              ssem).wait()

    def fill(i, carry):
        off = i * _VOCAB
        for v in range(_BPW // 16):
            sl = pl.ds(v * 16, 16)
            idxv[i, sl] = idxv[i, sl] + off
        return carry

    lax.fori_loop(0, _N_FIELDS, fill, 0)

    def start_gather(i, b):
        pltpu.async_copy(tab_hbm.at[idxv.at[i, pl.ds(b * _CHUNK, _CHUNK)]],
                         bufs[b], gsem[b])

    def wait_gather(b):
        pltpu.make_async_copy(tab_hbm.at[idxv.at[0, pl.ds(0, _CHUNK)]],
                              bufs[b], gsem[b]).wait()

    def out_slice(i, b):
        return out_hbm.at[pl.ds(row0 + b * _CHUNK, _CHUNK),
                          pl.ds(i * _EMB, _EMB)]

    for b in range(_NBUF):
        start_gather(0, b)

    def group(t, carry):
        for b in range(_NBUF):
            wait_gather(b)
            pltpu.async_copy(bufs[b], out_slice(t, b), wsem[b])
        for b in range(_NBUF):
            pltpu.make_async_copy(bufs[b], out_slice(0, 0), wsem[b]).wait()

            @pl.when(t + 1 < _N_FIELDS)
            def _():
                start_gather(t + 1, b)
        return carry

    lax.fori_loop(0, _N_FIELDS, group, 0)


@jax.jit
def _sc_gather(xt, tab):
    mesh = plsc.VectorSubcoreMesh(core_axis_name="c", subcore_axis_name="s")
    run = pl.kernel(
        _sc_body,
        mesh=mesh,
        out_type=jax.ShapeDtypeStruct((_BATCH, _N_FIELDS * _EMB), jnp.float32),
        compiler_params=pltpu.CompilerParams(use_tc_tiling_on_sc=True),
        scratch_types=(
            [pltpu.VMEM((_N_FIELDS, _BPW), jnp.int32)]
            + [pltpu.VMEM((_CHUNK, _EMB), jnp.float32)] * _NBUF
            + [pltpu.SemaphoreType.DMA] * (2 * _NBUF + 1)
        ),
    )
    return run(xt, tab)


def kernel(x, W):
    xt = x.astype(jnp.int32).T.reshape(_BATCH * _N_FIELDS)
    tab = W.reshape(_N_FIELDS * _VOCAB, _EMB)
    return _sc_gather(xt, tab)


# speedup vs baseline: 1.0043x; 1.0043x over previous
import jax
import jax.numpy as jnp
from jax import lax
from jax.experimental import pallas as pl
from jax.experimental.pallas import tpu as pltpu
from jax.experimental.pallas import tpu_sc as plsc

_N_FIELDS = 26
_VOCAB = 1000
_EMB = 128
_BATCH = 16384

_NC = 2
_NS = 16
_NW = _NC * _NS
_BPW = _BATCH // _NW
_CHUNK = 64
_NBUF = 8
_NQ = _BPW // _CHUNK


def _sc_body(xt_hbm, tab_hbm, out_hbm, idxv, *rest):
    bufs = rest[:_NBUF]
    ssem = rest[_NBUF]
    gsem = rest[_NBUF + 1:2 * _NBUF + 1]
    wsem = rest[2 * _NBUF + 1:]

    wid = lax.axis_index("s") * _NC + lax.axis_index("c")
    row0 = pl.multiple_of(wid * _BPW, _BPW)

    with jax.named_scope("idx_prep"):
        for i in range(_N_FIELDS):
            pltpu.async_copy(xt_hbm.at[pl.ds(i * _BATCH + row0, _BPW)],
                             idxv.at[i], ssem)
        for i in range(_N_FIELDS):
            pltpu.make_async_copy(xt_hbm.at[pl.ds(0, _BPW)], idxv.at[0],
                                  ssem).wait()

        def fill(i, carry):
            off = i * _VOCAB
            for v in range(_BPW // 16):
                sl = pl.ds(v * 16, 16)
                idxv[i, sl] = idxv[i, sl] + off
            return carry

        lax.fori_loop(0, _N_FIELDS, fill, 0)

    def start_gather(i, b):
        pltpu.async_copy(tab_hbm.at[idxv.at[i, pl.ds(b * _CHUNK, _CHUNK)]],
                         bufs[b], gsem[b])

    def wait_gather(b):
        pltpu.make_async_copy(tab_hbm.at[idxv.at[0, pl.ds(0, _CHUNK)]],
                              bufs[b], gsem[b]).wait()

    def out_slice(i, b):
        return out_hbm.at[pl.ds(row0 + b * _CHUNK, _CHUNK),
                          pl.ds(i * _EMB, _EMB)]

    with jax.named_scope("ring"):
        for b in range(_NBUF):
            start_gather(0, b)

        def group(t, carry):
            for b in range(_NBUF):
                wait_gather(b)
                pltpu.async_copy(bufs[b], out_slice(t, b), wsem[b])
            for b in range(_NBUF):
                pltpu.make_async_copy(bufs[b], out_slice(0, 0), wsem[b]).wait()

                @pl.when(t + 1 < _N_FIELDS)
                def _():
                    start_gather(t + 1, b)
            return carry

        lax.fori_loop(0, _N_FIELDS, group, 0)


@jax.jit
def _sc_gather(xt, tab):
    mesh = plsc.VectorSubcoreMesh(core_axis_name="c", subcore_axis_name="s")
    run = pl.kernel(
        _sc_body,
        mesh=mesh,
        out_type=jax.ShapeDtypeStruct((_BATCH, _N_FIELDS * _EMB), jnp.float32),
        compiler_params=pltpu.CompilerParams(use_tc_tiling_on_sc=True),
        scratch_types=(
            [pltpu.VMEM((_N_FIELDS, _BPW), jnp.int32)]
            + [pltpu.VMEM((_CHUNK, _EMB), jnp.float32)] * _NBUF
            + [pltpu.SemaphoreType.DMA] * (2 * _NBUF + 1)
        ),
    )
    return run(xt, tab)


def kernel(x, W):
    xt = x.astype(jnp.int32).T.reshape(_BATCH * _N_FIELDS)
    tab = W.reshape(_N_FIELDS * _VOCAB, _EMB)
    return _sc_gather(xt, tab)
